# Initial kernel scaffold; baseline (speedup 1.0000x reference)
import functools

import jax
import jax.numpy as jnp
from jax import lax
from jax.experimental import pallas as pl
from jax.experimental.pallas import tpu as pltpu
from jax.experimental.pallas import tpu_sc as plsc

_N = 10000
_D = 128
_E = 320000
_NC = 2
_NS = 16
_NW = _NC * _NS
_CE = 128
_CH = 80
_EPAD = _NW * _CH * _CE
_RPW = _N // _NS
_RCP = 125
_L = 16


def _msg_body(y_hbm, src_hbm, dst_hbm, w_hbm, out_hbm,
              src_v, dst_v, w_v, rows_v, acc):
    c = lax.axis_index("c")
    s = lax.axis_index("s")
    wid = c * _NS + s

    z16 = jnp.zeros((_L,), jnp.float32)

    def zrow(r, carry):
        for k in range(_D // _L):
            rows_v[r, pl.ds(k * _L, _L)] = z16
        return carry

    lax.fori_loop(0, _CE, zrow, 0)

    for t in range(_RPW // _RCP):
        off = s * _RPW + t * _RCP
        pltpu.sync_copy(rows_v.at[pl.ds(0, _RCP)], acc.at[pl.ds(off, _RCP)])
    plsc.subcore_barrier()

    pltpu.sync_copy(src_hbm.at[wid], src_v)
    pltpu.sync_copy(dst_hbm.at[wid], dst_v)
    pltpu.sync_copy(w_hbm.at[wid], w_v)

    def chunk(j, carry):
        pltpu.sync_copy(y_hbm.at[src_v.at[j]], rows_v)

        def srow(r, carry2):
            jj = jnp.full((_L,), j, jnp.int32)
            rr = jnp.full((_L,), r, jnp.int32)
            wb = plsc.load_gather(w_v, [jj, rr])
            for k in range(_D // _L):
                sl = pl.ds(k * _L, _L)
                rows_v[r, sl] = rows_v[r, sl] * wb
            return carry2

        lax.fori_loop(0, _CE, srow, 0)

        pltpu.sync_copy(rows_v, acc.at[dst_v.at[j]], add=True)
        return carry

    lax.fori_loop(0, _CH, chunk, 0)

    plsc.subcore_barrier()
    for t in range(_RPW // _RCP):
        off = s * _RPW + t * _RCP
        pltpu.sync_copy(acc.at[pl.ds(off, _RCP)], rows_v.at[pl.ds(0, _RCP)])
        pltpu.sync_copy(rows_v.at[pl.ds(0, _RCP)],
                        out_hbm.at[c, pl.ds(off, _RCP)])


_msg_call = pl.kernel(
    _msg_body,
    out_type=jax.ShapeDtypeStruct((_NC, _N, _D), jnp.float32),
    mesh=plsc.VectorSubcoreMesh(core_axis_name="c", subcore_axis_name="s"),
    scratch_types=[
        pltpu.VMEM((_CH, _CE), jnp.int32),
        pltpu.VMEM((_CH, _CE), jnp.int32),
        pltpu.VMEM((_CH, _CE), jnp.float32),
        pltpu.VMEM((_CE, _D), jnp.float32),
        pltpu.VMEM_SHARED((_N, _D), jnp.float32),
    ],
)


_R = 1000


def _enc_body(x_ref, p0_ref, p1_ref, ew1, eb1, ew2, eb2, gw, gb,
              y_ref, self_ref, dinv_ref, invd_ref):
    x = x_ref[...]
    h = jnp.maximum(
        jnp.dot(x, ew1[...], preferred_element_type=jnp.float32) + eb1[...],
        0.0)
    h = jnp.dot(h, ew2[...], preferred_element_type=jnp.float32) + eb2[...]
    deg = p0_ref[:, :1] + p1_ref[:, :1] + 1.0
    dinv = lax.rsqrt(deg)
    invd = 1.0 / deg
    xw = jnp.dot(h, gw[...], preferred_element_type=jnp.float32)
    y_ref[...] = xw * dinv
    self_ref[...] = xw * invd + gb[...]
    dinv_ref[...] = jnp.broadcast_to(dinv, dinv_ref.shape)
    invd_ref[...] = jnp.broadcast_to(invd, invd_ref.shape)


_enc_call = pl.pallas_call(
    _enc_body,
    grid=(_N // _R,),
    in_specs=[
        pl.BlockSpec((_R, 2), lambda i: (i, 0)),
        pl.BlockSpec((_R, _D), lambda i: (i, 0)),
        pl.BlockSpec((_R, _D), lambda i: (i, 0)),
        pl.BlockSpec((2, _D), lambda i: (0, 0)),
        pl.BlockSpec((1, _D), lambda i: (0, 0)),
        pl.BlockSpec((_D, _D), lambda i: (0, 0)),
        pl.BlockSpec((1, _D), lambda i: (0, 0)),
        pl.BlockSpec((_D, _D), lambda i: (0, 0)),
        pl.BlockSpec((1, _D), lambda i: (0, 0)),
    ],
    out_specs=[
        pl.BlockSpec((_R, _D), lambda i: (i, 0)),
        pl.BlockSpec((_R, _D), lambda i: (i, 0)),
        pl.BlockSpec((_R, _D), lambda i: (i, 0)),
        pl.BlockSpec((_R, _D), lambda i: (i, 0)),
    ],
    out_shape=[jax.ShapeDtypeStruct((_N, _D), jnp.float32)] * 4,
)


def _mid_body(p0_ref, p1_ref, selfp_ref, dinv_ref, invd_ref, w_ref, b_ref,
              y_ref, selfn_ref):
    dinv = dinv_ref[...]
    h = jnp.maximum((p0_ref[...] + p1_ref[...]) * dinv + selfp_ref[...], 0.0)
    xw = jnp.dot(h, w_ref[...], preferred_element_type=jnp.float32)
    y_ref[...] = xw * dinv
    selfn_ref[...] = xw * invd_ref[...] + b_ref[...]


_mid_call = pl.pallas_call(
    _mid_body,
    grid=(_N // _R,),
    in_specs=[
        pl.BlockSpec((_R, _D), lambda i: (i, 0)),
        pl.BlockSpec((_R, _D), lambda i: (i, 0)),
        pl.BlockSpec((_R, _D), lambda i: (i, 0)),
        pl.BlockSpec((_R, _D), lambda i: (i, 0)),
        pl.BlockSpec((_R, _D), lambda i: (i, 0)),
        pl.BlockSpec((_D, _D), lambda i: (0, 0)),
        pl.BlockSpec((1, _D), lambda i: (0, 0)),
    ],
    out_specs=[
        pl.BlockSpec((_R, _D), lambda i: (i, 0)),
        pl.BlockSpec((_R, _D), lambda i: (i, 0)),
    ],
    out_shape=[jax.ShapeDtypeStruct((_N, _D), jnp.float32)] * 2,
)


def _dec_body(p0_ref, p1_ref, self2_ref, dinv_ref, x_ref,
              w1_ref, b1_ref, w2_ref, b2_ref, out_ref):
    h = jnp.maximum(
        (p0_ref[...] + p1_ref[...]) * dinv_ref[...] + self2_ref[...], 0.0)
    t = jnp.maximum(
        jnp.dot(h, w1_ref[...], preferred_element_type=jnp.float32)
        + b1_ref[...], 0.0)
    p = jnp.dot(t, w2_ref[...], preferred_element_type=jnp.float32) + b2_ref[...]
    m = jnp.max(p, axis=1, keepdims=True)
    e = jnp.exp(p - m)
    sm = e / jnp.sum(e, axis=1, keepdims=True)
    out_ref[...] = sm + x_ref[...] * jnp.array([[2.0, 0.0]], jnp.float32)


_dec_call = pl.pallas_call(
    _dec_body,
    grid=(_N // _R,),
    in_specs=[
        pl.BlockSpec((_R, _D), lambda i: (i, 0)),
        pl.BlockSpec((_R, _D), lambda i: (i, 0)),
        pl.BlockSpec((_R, _D), lambda i: (i, 0)),
        pl.BlockSpec((_R, _D), lambda i: (i, 0)),
        pl.BlockSpec((_R, 2), lambda i: (i, 0)),
        pl.BlockSpec((_D, _D), lambda i: (0, 0)),
        pl.BlockSpec((1, _D), lambda i: (0, 0)),
        pl.BlockSpec((_D, 2), lambda i: (0, 0)),
        pl.BlockSpec((1, 2), lambda i: (0, 0)),
    ],
    out_specs=pl.BlockSpec((_R, 2), lambda i: (i, 0)),
    out_shape=jax.ShapeDtypeStruct((_N, 2), jnp.float32),
)


def kernel(x, edge_index, edge_weight, enc_W1, enc_b1, enc_W2, enc_b2,
           gcn_W0, gcn_b0, gcn_W1, gcn_b1, gcn_W2, gcn_b2,
           dec_W1, dec_b1, dec_W2, dec_b2):
    src = edge_index[0].astype(jnp.int32)
    dst = edge_index[1].astype(jnp.int32)
    w = edge_weight.astype(jnp.float32)
    pad = _EPAD - _E
    src3 = jnp.concatenate([src, jnp.zeros((pad,), jnp.int32)]).reshape(
        _NW, _CH, _CE)
    dst3 = jnp.concatenate([dst, jnp.zeros((pad,), jnp.int32)]).reshape(
        _NW, _CH, _CE)
    w3 = jnp.concatenate([w, jnp.zeros((pad,), jnp.float32)]).reshape(
        _NW, _CH, _CE)

    ones = jnp.ones((_N, _D), jnp.float32)
    degp = _msg_call(ones, src3, dst3, w3)

    y0, self0, dinvB, invdB = _enc_call(
        x, degp[0], degp[1], enc_W1, enc_b1.reshape(1, -1), enc_W2,
        enc_b2.reshape(1, -1), gcn_W0, gcn_b0.reshape(1, -1))
    p = _msg_call(y0, src3, dst3, w3)
    y1, self1 = _mid_call(p[0], p[1], self0, dinvB, invdB,
                          gcn_W1, gcn_b1.reshape(1, -1))
    p = _msg_call(y1, src3, dst3, w3)
    y2, self2 = _mid_call(p[0], p[1], self1, dinvB, invdB,
                          gcn_W2, gcn_b2.reshape(1, -1))
    p = _msg_call(y2, src3, dst3, w3)
    out = _dec_call(p[0], p[1], self2, dinvB, x,
                    dec_W1, dec_b1.reshape(1, -1), dec_W2,
                    dec_b2.reshape(1, -1))
    return out


# speedup vs baseline: 3.8645x; 3.8645x over previous
import functools

import jax
import jax.numpy as jnp
from jax import lax
from jax.experimental import pallas as pl
from jax.experimental.pallas import tpu as pltpu
from jax.experimental.pallas import tpu_sc as plsc

_N = 10000
_D = 128
_E = 320000
_NC = 2
_NS = 16
_NW = _NC * _NS
_CE = 128
_CH = 80
_EPAD = _NW * _CH * _CE
_NP = 10240
_RPW = _NP // _NS
_RCP = 128
_L = 16


def _msg_body(y_hbm, src_hbm, dst_hbm, w_hbm, out_hbm,
              src_v, dst_v, w_v, rows_v, acc):
    c = lax.axis_index("c")
    s = lax.axis_index("s")
    wid = c * _NS + s

    z16 = jnp.zeros((_L,), jnp.float32)

    def zrow(r, carry):
        for k in range(_D // _L):
            rows_v[r, pl.ds(k * _L, _L)] = z16
        return carry

    lax.fori_loop(0, _CE, zrow, 0)

    for t in range(_RPW // _RCP):
        off = s * _RPW + t * _RCP
        pltpu.sync_copy(rows_v.at[pl.ds(0, _RCP)], acc.at[pl.ds(off, _RCP)])
    plsc.subcore_barrier()

    pltpu.sync_copy(src_hbm.at[wid], src_v)
    pltpu.sync_copy(dst_hbm.at[wid], dst_v)
    pltpu.sync_copy(w_hbm.at[wid], w_v)

    def chunk(j, carry):
        pltpu.sync_copy(y_hbm.at[src_v.at[j]], rows_v)

        def srow(r, carry2):
            ii = jnp.full((_L,), j * _CE + r, jnp.int32)
            wb = plsc.load_gather(w_v, [ii])
            for k in range(_D // _L):
                sl = pl.ds(k * _L, _L)
                rows_v[r, sl] = rows_v[r, sl] * wb
            return carry2

        lax.fori_loop(0, _CE, srow, 0)

        pltpu.sync_copy(rows_v, acc.at[dst_v.at[j]], add=True)
        return carry

    lax.fori_loop(0, _CH, chunk, 0)

    plsc.subcore_barrier()
    for t in range(_RPW // _RCP):
        off = s * _RPW + t * _RCP
        pltpu.sync_copy(acc.at[pl.ds(off, _RCP)], rows_v.at[pl.ds(0, _RCP)])
        pltpu.sync_copy(rows_v.at[pl.ds(0, _RCP)],
                        out_hbm.at[c, pl.ds(off, _RCP)])


_msg_call = pl.kernel(
    _msg_body,
    out_type=jax.ShapeDtypeStruct((_NC, _NP, _D), jnp.float32),
    mesh=plsc.VectorSubcoreMesh(core_axis_name="c", subcore_axis_name="s"),
    compiler_params=pltpu.CompilerParams(needs_layout_passes=False),
    scratch_types=[
        pltpu.VMEM((_CH, _CE), jnp.int32),
        pltpu.VMEM((_CH, _CE), jnp.int32),
        pltpu.VMEM((_CH * _CE,), jnp.float32),
        pltpu.VMEM((_CE, _D), jnp.float32),
        pltpu.VMEM_SHARED((_NP, _D), jnp.float32),
    ],
)


_R = 1000


def _enc_body(x_ref, p0_ref, p1_ref, ew1, eb1, ew2, eb2, gw, gb,
              y_ref, self_ref, dinv_ref, invd_ref):
    x = x_ref[...]
    h = jnp.maximum(
        jnp.dot(x, ew1[...], preferred_element_type=jnp.float32) + eb1[...],
        0.0)
    h = jnp.dot(h, ew2[...], preferred_element_type=jnp.float32) + eb2[...]
    deg = p0_ref[:, :1] + p1_ref[:, :1] + 1.0
    dinv = lax.rsqrt(deg)
    invd = 1.0 / deg
    xw = jnp.dot(h, gw[...], preferred_element_type=jnp.float32)
    y_ref[...] = xw * dinv
    self_ref[...] = xw * invd + gb[...]
    dinv_ref[...] = jnp.broadcast_to(dinv, dinv_ref.shape)
    invd_ref[...] = jnp.broadcast_to(invd, invd_ref.shape)


_enc_call = pl.pallas_call(
    _enc_body,
    grid=(_N // _R,),
    in_specs=[
        pl.BlockSpec((_R, 2), lambda i: (i, 0)),
        pl.BlockSpec((_R, _D), lambda i: (i, 0)),
        pl.BlockSpec((_R, _D), lambda i: (i, 0)),
        pl.BlockSpec((2, _D), lambda i: (0, 0)),
        pl.BlockSpec((1, _D), lambda i: (0, 0)),
        pl.BlockSpec((_D, _D), lambda i: (0, 0)),
        pl.BlockSpec((1, _D), lambda i: (0, 0)),
        pl.BlockSpec((_D, _D), lambda i: (0, 0)),
        pl.BlockSpec((1, _D), lambda i: (0, 0)),
    ],
    out_specs=[
        pl.BlockSpec((_R, _D), lambda i: (i, 0)),
        pl.BlockSpec((_R, _D), lambda i: (i, 0)),
        pl.BlockSpec((_R, _D), lambda i: (i, 0)),
        pl.BlockSpec((_R, _D), lambda i: (i, 0)),
    ],
    out_shape=[jax.ShapeDtypeStruct((_N, _D), jnp.float32)] * 4,
)


def _mid_body(p0_ref, p1_ref, selfp_ref, dinv_ref, invd_ref, w_ref, b_ref,
              y_ref, selfn_ref):
    dinv = dinv_ref[...]
    h = jnp.maximum((p0_ref[...] + p1_ref[...]) * dinv + selfp_ref[...], 0.0)
    xw = jnp.dot(h, w_ref[...], preferred_element_type=jnp.float32)
    y_ref[...] = xw * dinv
    selfn_ref[...] = xw * invd_ref[...] + b_ref[...]


_mid_call = pl.pallas_call(
    _mid_body,
    grid=(_N // _R,),
    in_specs=[
        pl.BlockSpec((_R, _D), lambda i: (i, 0)),
        pl.BlockSpec((_R, _D), lambda i: (i, 0)),
        pl.BlockSpec((_R, _D), lambda i: (i, 0)),
        pl.BlockSpec((_R, _D), lambda i: (i, 0)),
        pl.BlockSpec((_R, _D), lambda i: (i, 0)),
        pl.BlockSpec((_D, _D), lambda i: (0, 0)),
        pl.BlockSpec((1, _D), lambda i: (0, 0)),
    ],
    out_specs=[
        pl.BlockSpec((_R, _D), lambda i: (i, 0)),
        pl.BlockSpec((_R, _D), lambda i: (i, 0)),
    ],
    out_shape=[jax.ShapeDtypeStruct((_N, _D), jnp.float32)] * 2,
)


def _dec_body(p0_ref, p1_ref, self2_ref, dinv_ref, x_ref,
              w1_ref, b1_ref, w2_ref, b2_ref, out_ref):
    h = jnp.maximum(
        (p0_ref[...] + p1_ref[...]) * dinv_ref[...] + self2_ref[...], 0.0)
    t = jnp.maximum(
        jnp.dot(h, w1_ref[...], preferred_element_type=jnp.float32)
        + b1_ref[...], 0.0)
    p = jnp.dot(t, w2_ref[...], preferred_element_type=jnp.float32) + b2_ref[...]
    m = jnp.max(p, axis=1, keepdims=True)
    e = jnp.exp(p - m)
    sm = e / jnp.sum(e, axis=1, keepdims=True)
    col = lax.broadcasted_iota(jnp.int32, (1, 2), 1)
    wc = jnp.where(col == 0, 2.0, 0.0).astype(jnp.float32)
    out_ref[...] = sm + x_ref[...] * wc


_dec_call = pl.pallas_call(
    _dec_body,
    grid=(_N // _R,),
    in_specs=[
        pl.BlockSpec((_R, _D), lambda i: (i, 0)),
        pl.BlockSpec((_R, _D), lambda i: (i, 0)),
        pl.BlockSpec((_R, _D), lambda i: (i, 0)),
        pl.BlockSpec((_R, _D), lambda i: (i, 0)),
        pl.BlockSpec((_R, 2), lambda i: (i, 0)),
        pl.BlockSpec((_D, _D), lambda i: (0, 0)),
        pl.BlockSpec((1, _D), lambda i: (0, 0)),
        pl.BlockSpec((_D, 2), lambda i: (0, 0)),
        pl.BlockSpec((1, 2), lambda i: (0, 0)),
    ],
    out_specs=pl.BlockSpec((_R, 2), lambda i: (i, 0)),
    out_shape=jax.ShapeDtypeStruct((_N, 2), jnp.float32),
)


def kernel(x, edge_index, edge_weight, enc_W1, enc_b1, enc_W2, enc_b2,
           gcn_W0, gcn_b0, gcn_W1, gcn_b1, gcn_W2, gcn_b2,
           dec_W1, dec_b1, dec_W2, dec_b2):
    src = edge_index[0].astype(jnp.int32)
    dst = edge_index[1].astype(jnp.int32)
    w = edge_weight.astype(jnp.float32)
    pad = _EPAD - _E
    src3 = jnp.concatenate([src, jnp.zeros((pad,), jnp.int32)]).reshape(
        _NW, _CH, _CE)
    dst3 = jnp.concatenate([dst, jnp.zeros((pad,), jnp.int32)]).reshape(
        _NW, _CH, _CE)
    w3 = jnp.concatenate([w, jnp.zeros((pad,), jnp.float32)]).reshape(
        _NW, _CH * _CE)

    ones = jnp.ones((_N, _D), jnp.float32)
    degp = _msg_call(ones, src3, dst3, w3)

    y0, self0, dinvB, invdB = _enc_call(
        x, degp[0, :_N], degp[1, :_N], enc_W1, enc_b1.reshape(1, -1), enc_W2,
        enc_b2.reshape(1, -1), gcn_W0, gcn_b0.reshape(1, -1))
    p = _msg_call(y0, src3, dst3, w3)
    y1, self1 = _mid_call(p[0, :_N], p[1, :_N], self0, dinvB, invdB,
                          gcn_W1, gcn_b1.reshape(1, -1))
    p = _msg_call(y1, src3, dst3, w3)
    y2, self2 = _mid_call(p[0, :_N], p[1, :_N], self1, dinvB, invdB,
                          gcn_W2, gcn_b2.reshape(1, -1))
    p = _msg_call(y2, src3, dst3, w3)
    out = _dec_call(p[0, :_N], p[1, :_N], self2, dinvB, x,
                    dec_W1, dec_b1.reshape(1, -1), dec_W2,
                    dec_b2.reshape(1, -1))
    return out


# speedup vs baseline: 6.2672x; 1.6217x over previous
import functools

import jax
import jax.numpy as jnp
from jax import lax
from jax.experimental import pallas as pl
from jax.experimental.pallas import tpu as pltpu
from jax.experimental.pallas import tpu_sc as plsc

_N = 10000
_D = 128
_E = 320000
_NC = 2
_NS = 16
_NW = _NC * _NS
_CE = 128
_CH = 80
_GC = 16
_EPAD = _NW * _CH * _CE
_NP = 10240
_RPW = _NP // _NS
_RCP = 128
_L = 16


def _msg_body(y_hbm, src_hbm, dst_hbm, w_hbm, out_hbm,
              src_v, dst_v, w_v, rows_v, acc,
              gsem0, gsem1, ssem0, ssem1):
    c = lax.axis_index("c")
    s = lax.axis_index("s")
    wid = c * _NS + s

    z16 = jnp.zeros((_L,), jnp.float32)

    def zrow(r, carry):
        for k in range(_D // _L):
            rows_v[0, r, pl.ds(k * _L, _L)] = z16
        return carry

    lax.fori_loop(0, _CE, zrow, 0)

    for t in range(_RPW // _RCP):
        off = s * _RPW + t * _RCP
        pltpu.sync_copy(rows_v.at[0, pl.ds(0, _RCP)],
                        acc.at[pl.ds(off, _RCP)])
    plsc.subcore_barrier()

    gsems = (gsem0, gsem1)
    ssems = (ssem0, ssem1)

    def _start_gather(b, j):
        pltpu.async_copy(y_hbm.at[src_v.at[j]], rows_v.at[b], gsems[b])

    def _wait_gather(b):
        pltpu.make_async_copy(y_hbm.at[src_v.at[0]], rows_v.at[b],
                              gsems[b]).wait()

    def _start_scatter(b, j):
        pltpu.async_copy(rows_v.at[b], acc.at[dst_v.at[j]], ssems[b],
                         add=True)

    def _wait_scatter(b):
        pltpu.make_async_copy(rows_v.at[b], acc.at[dst_v.at[0]],
                              ssems[b]).wait()

    def _scale(b, j):
        def srow(r, carry2):
            ii = jnp.full((_L,), j * _CE + r, jnp.int32)
            wb = plsc.load_gather(w_v, [ii])
            for k in range(_D // _L):
                sl = pl.ds(k * _L, _L)
                rows_v[b, r, sl] = rows_v[b, r, sl] * wb
            return carry2

        lax.fori_loop(0, _CE, srow, 0)

    def group(g, carry):
        pltpu.sync_copy(src_hbm.at[wid, pl.ds(g * _GC, _GC)], src_v)
        pltpu.sync_copy(dst_hbm.at[wid, pl.ds(g * _GC, _GC)], dst_v)
        pltpu.sync_copy(w_hbm.at[wid, pl.ds(g * _GC * _CE, _GC * _CE)], w_v)

        _start_gather(0, 0)

        def pipe(jj, carry2):
            j0 = 2 * jj
            j1 = j0 + 1

            @pl.when(jj > 0)
            def _():
                _wait_scatter(1)

            _start_gather(1, j1)
            _wait_gather(0)
            _scale(0, j0)
            _start_scatter(0, j0)
            _wait_gather(1)
            _scale(1, j1)
            _start_scatter(1, j1)
            _wait_scatter(0)

            @pl.when(jj < _GC // 2 - 1)
            def _():
                _start_gather(0, j0 + 2)

            return carry2

        lax.fori_loop(0, _GC // 2, pipe, 0)
        _wait_scatter(1)
        return carry

    lax.fori_loop(0, _CH // _GC, group, 0)

    plsc.subcore_barrier()
    for t in range(_RPW // _RCP):
        off = s * _RPW + t * _RCP
        pltpu.sync_copy(acc.at[pl.ds(off, _RCP)],
                        rows_v.at[0, pl.ds(0, _RCP)])
        pltpu.sync_copy(rows_v.at[0, pl.ds(0, _RCP)],
                        out_hbm.at[c, pl.ds(off, _RCP)])


_msg_call = pl.kernel(
    _msg_body,
    out_type=jax.ShapeDtypeStruct((_NC, _NP, _D), jnp.float32),
    mesh=plsc.VectorSubcoreMesh(core_axis_name="c", subcore_axis_name="s"),
    compiler_params=pltpu.CompilerParams(needs_layout_passes=False),
    scratch_types=[
        pltpu.VMEM((_GC, _CE), jnp.int32),
        pltpu.VMEM((_GC, _CE), jnp.int32),
        pltpu.VMEM((_GC * _CE,), jnp.float32),
        pltpu.VMEM((2, _CE, _D), jnp.float32),
        pltpu.VMEM_SHARED((_NP, _D), jnp.float32),
        pltpu.SemaphoreType.DMA,
        pltpu.SemaphoreType.DMA,
        pltpu.SemaphoreType.DMA,
        pltpu.SemaphoreType.DMA,
    ],
)


def _deg_body(dst_hbm, w_hbm, out_hbm, dst_v, w_v, zb_v, acc1):
    c = lax.axis_index("c")
    s = lax.axis_index("s")
    wid = c * _NS + s

    z16 = jnp.zeros((_L,), jnp.float32)
    for k in range(_RPW // _L):
        zb_v[pl.ds(k * _L, _L)] = z16
    pltpu.sync_copy(zb_v, acc1.at[pl.ds(s * _RPW, _RPW)])
    plsc.subcore_barrier()

    pltpu.sync_copy(dst_hbm.at[wid], dst_v)
    pltpu.sync_copy(w_hbm.at[wid], w_v)

    def chunk(j, carry):
        pltpu.sync_copy(w_v.at[pl.ds(j * _CE, _CE)],
                        acc1.at[dst_v.at[j]], add=True)
        return carry

    lax.fori_loop(0, _CH, chunk, 0)

    plsc.subcore_barrier()
    pltpu.sync_copy(acc1.at[pl.ds(s * _RPW, _RPW)],
                    out_hbm.at[c, pl.ds(s * _RPW, _RPW)])


_deg_call = pl.kernel(
    _deg_body,
    out_type=jax.ShapeDtypeStruct((_NC, _NP), jnp.float32),
    mesh=plsc.VectorSubcoreMesh(core_axis_name="c", subcore_axis_name="s"),
    compiler_params=pltpu.CompilerParams(needs_layout_passes=False),
    scratch_types=[
        pltpu.VMEM((_CH, _CE), jnp.int32),
        pltpu.VMEM((_CH * _CE,), jnp.float32),
        pltpu.VMEM((_RPW,), jnp.float32),
        pltpu.VMEM_SHARED((_NP,), jnp.float32),
    ],
)


_R = 1000


def _enc_body(x_ref, dp_ref, ew1, eb1, ew2, eb2, gw, gb,
              y_ref, self_ref, dinv_ref, invd_ref):
    x = x_ref[...]
    h = jnp.maximum(
        jnp.dot(x, ew1[...], preferred_element_type=jnp.float32) + eb1[...],
        0.0)
    h = jnp.dot(h, ew2[...], preferred_element_type=jnp.float32) + eb2[...]
    deg = dp_ref[:, 0:1] + dp_ref[:, 1:2] + 1.0
    dinv = lax.rsqrt(deg)
    invd = 1.0 / deg
    xw = jnp.dot(h, gw[...], preferred_element_type=jnp.float32)
    y_ref[...] = xw * dinv
    self_ref[...] = xw * invd + gb[...]
    dinv_ref[...] = jnp.broadcast_to(dinv, dinv_ref.shape)
    invd_ref[...] = jnp.broadcast_to(invd, invd_ref.shape)


_enc_call = pl.pallas_call(
    _enc_body,
    grid=(_N // _R,),
    in_specs=[
        pl.BlockSpec((_R, 2), lambda i: (i, 0)),
        pl.BlockSpec((_R, 2), lambda i: (i, 0)),
        pl.BlockSpec((2, _D), lambda i: (0, 0)),
        pl.BlockSpec((1, _D), lambda i: (0, 0)),
        pl.BlockSpec((_D, _D), lambda i: (0, 0)),
        pl.BlockSpec((1, _D), lambda i: (0, 0)),
        pl.BlockSpec((_D, _D), lambda i: (0, 0)),
        pl.BlockSpec((1, _D), lambda i: (0, 0)),
    ],
    out_specs=[
        pl.BlockSpec((_R, _D), lambda i: (i, 0)),
        pl.BlockSpec((_R, _D), lambda i: (i, 0)),
        pl.BlockSpec((_R, _D), lambda i: (i, 0)),
        pl.BlockSpec((_R, _D), lambda i: (i, 0)),
    ],
    out_shape=[jax.ShapeDtypeStruct((_N, _D), jnp.float32)] * 4,
)


def _mid_body(p0_ref, p1_ref, selfp_ref, dinv_ref, invd_ref, w_ref, b_ref,
              y_ref, selfn_ref):
    dinv = dinv_ref[...]
    h = jnp.maximum((p0_ref[...] + p1_ref[...]) * dinv + selfp_ref[...], 0.0)
    xw = jnp.dot(h, w_ref[...], preferred_element_type=jnp.float32)
    y_ref[...] = xw * dinv
    selfn_ref[...] = xw * invd_ref[...] + b_ref[...]


_mid_call = pl.pallas_call(
    _mid_body,
    grid=(_N // _R,),
    in_specs=[
        pl.BlockSpec((_R, _D), lambda i: (i, 0)),
        pl.BlockSpec((_R, _D), lambda i: (i, 0)),
        pl.BlockSpec((_R, _D), lambda i: (i, 0)),
        pl.BlockSpec((_R, _D), lambda i: (i, 0)),
        pl.BlockSpec((_R, _D), lambda i: (i, 0)),
        pl.BlockSpec((_D, _D), lambda i: (0, 0)),
        pl.BlockSpec((1, _D), lambda i: (0, 0)),
    ],
    out_specs=[
        pl.BlockSpec((_R, _D), lambda i: (i, 0)),
        pl.BlockSpec((_R, _D), lambda i: (i, 0)),
    ],
    out_shape=[jax.ShapeDtypeStruct((_N, _D), jnp.float32)] * 2,
)


def _dec_body(p0_ref, p1_ref, self2_ref, dinv_ref, x_ref,
              w1_ref, b1_ref, w2_ref, b2_ref, out_ref):
    h = jnp.maximum(
        (p0_ref[...] + p1_ref[...]) * dinv_ref[...] + self2_ref[...], 0.0)
    t = jnp.maximum(
        jnp.dot(h, w1_ref[...], preferred_element_type=jnp.float32)
        + b1_ref[...], 0.0)
    p = jnp.dot(t, w2_ref[...], preferred_element_type=jnp.float32) + b2_ref[...]
    m = jnp.max(p, axis=1, keepdims=True)
    e = jnp.exp(p - m)
    sm = e / jnp.sum(e, axis=1, keepdims=True)
    col = lax.broadcasted_iota(jnp.int32, (1, 2), 1)
    wc = jnp.where(col == 0, 2.0, 0.0).astype(jnp.float32)
    out_ref[...] = sm + x_ref[...] * wc


_dec_call = pl.pallas_call(
    _dec_body,
    grid=(_N // _R,),
    in_specs=[
        pl.BlockSpec((_R, _D), lambda i: (i, 0)),
        pl.BlockSpec((_R, _D), lambda i: (i, 0)),
        pl.BlockSpec((_R, _D), lambda i: (i, 0)),
        pl.BlockSpec((_R, _D), lambda i: (i, 0)),
        pl.BlockSpec((_R, 2), lambda i: (i, 0)),
        pl.BlockSpec((_D, _D), lambda i: (0, 0)),
        pl.BlockSpec((1, _D), lambda i: (0, 0)),
        pl.BlockSpec((_D, 2), lambda i: (0, 0)),
        pl.BlockSpec((1, 2), lambda i: (0, 0)),
    ],
    out_specs=pl.BlockSpec((_R, 2), lambda i: (i, 0)),
    out_shape=jax.ShapeDtypeStruct((_N, 2), jnp.float32),
)


def kernel(x, edge_index, edge_weight, enc_W1, enc_b1, enc_W2, enc_b2,
           gcn_W0, gcn_b0, gcn_W1, gcn_b1, gcn_W2, gcn_b2,
           dec_W1, dec_b1, dec_W2, dec_b2):
    src = edge_index[0].astype(jnp.int32)
    dst = edge_index[1].astype(jnp.int32)
    w = edge_weight.astype(jnp.float32)
    pad = _EPAD - _E
    src3 = jnp.concatenate([src, jnp.zeros((pad,), jnp.int32)]).reshape(
        _NW, _CH, _CE)
    dst3 = jnp.concatenate([dst, jnp.zeros((pad,), jnp.int32)]).reshape(
        _NW, _CH, _CE)
    w3 = jnp.concatenate([w, jnp.zeros((pad,), jnp.float32)]).reshape(
        _NW, _CH * _CE)

    degp = _deg_call(dst3, w3)
    dp = jnp.transpose(degp[:, :_N])

    y0, self0, dinvB, invdB = _enc_call(
        x, dp, enc_W1, enc_b1.reshape(1, -1), enc_W2,
        enc_b2.reshape(1, -1), gcn_W0, gcn_b0.reshape(1, -1))
    p = _msg_call(y0, src3, dst3, w3)
    y1, self1 = _mid_call(p[0, :_N], p[1, :_N], self0, dinvB, invdB,
                          gcn_W1, gcn_b1.reshape(1, -1))
    p = _msg_call(y1, src3, dst3, w3)
    y2, self2 = _mid_call(p[0, :_N], p[1, :_N], self1, dinvB, invdB,
                          gcn_W2, gcn_b2.reshape(1, -1))
    p = _msg_call(y2, src3, dst3, w3)
    out = _dec_call(p[0, :_N], p[1, :_N], self2, dinvB, x,
                    dec_W1, dec_b1.reshape(1, -1), dec_W2,
                    dec_b2.reshape(1, -1))
    return out
